# baseline (speedup 1.0000x reference)
import functools

import jax
import jax.numpy as jnp
from jax import lax
from jax.experimental import pallas as pl
from jax.experimental.pallas import tpu as pltpu
from jax.experimental.pallas import tpu_sc as plsc

_NBL = 64
_NANT = 65
_NT = 128
_NF = 4096
_COLS = _NT * _NF
_NW = 32
_CW = _COLS // _NW
_CHW = 8192
_L = 16


def _mul3(dst, a, b, c, n):
    @plsc.parallel_loop(0, n, step=_L, unroll=8)
    def _body(i):
        sl = pl.ds(i, _L)
        dst[sl] = a[sl] * b[sl] * c[sl]


def _sc_body(vm_hbm, j_hbm, out_hbm, jb, vmb, ob, jsem, vsem, osem):
    c = lax.axis_index("c")
    s = lax.axis_index("s")
    wid = s * 2 + c
    col0 = wid * _CW

    for cc in range(_CW // _CHW):
        cb = col0 + cc * _CHW

        def jsrc(r):
            return j_hbm.at[r, pl.ds(cb, _CHW)]

        def vsrc(r):
            return vm_hbm.at[r, pl.ds(cb, _CHW)]

        def odst(r):
            return out_hbm.at[r, pl.ds(cb, _CHW)]

        pltpu.sync_copy(jsrc(0), jb.at[0])
        pltpu.async_copy(jsrc(1), jb.at[1], jsem.at[1])
        pltpu.async_copy(jsrc(2), jb.at[2], jsem.at[2])
        pltpu.async_copy(vsrc(0), vmb.at[0], vsem.at[0])
        pltpu.async_copy(vsrc(1), vmb.at[1], vsem.at[1])

        def block(k, _):
            for q in range(4):
                r = 4 * k + q
                jL = jb.at[q]
                jR = jb.at[(q + 1) % 4]
                vcur = vmb.at[q]
                ocur = ob.at[q]

                @pl.when(r <= _NANT - 4)
                def _pj():
                    pltpu.async_copy(jsrc(r + 3), jb.at[(q + 3) % 4],
                                     jsem.at[(q + 3) % 4])

                @pl.when(r <= _NBL - 3)
                def _pv():
                    pltpu.async_copy(vsrc(r + 2), vmb.at[(q + 2) % 4],
                                     vsem.at[(q + 2) % 4])

                pltpu.make_async_copy(jsrc(r + 1), jR,
                                      jsem.at[(q + 1) % 4]).wait()
                pltpu.make_async_copy(vsrc(r), vcur, vsem.at[q]).wait()

                @pl.when(r >= 4)
                def _po():
                    pltpu.make_async_copy(ocur, odst(r), osem.at[q]).wait()

                _mul3(ocur, jL, vcur, jR, _CHW)
                pltpu.async_copy(ocur, odst(r), osem.at[q])
            return 0

        lax.fori_loop(0, _NBL // 4, block, 0)
        for q in range(4):
            pltpu.make_async_copy(ob.at[q], odst(q), osem.at[q]).wait()


def kernel(V_m, jones):
    vm2 = V_m.reshape(_NBL, _COLS)
    j2 = jones.reshape(_NANT, _COLS)
    mesh = plsc.VectorSubcoreMesh(core_axis_name="c", subcore_axis_name="s")
    run = functools.partial(
        pl.kernel,
        mesh=mesh,
        out_type=jax.ShapeDtypeStruct((_NBL, _COLS), jnp.float32),
        scratch_types=[
            pltpu.VMEM((4, _CHW), jnp.float32),
            pltpu.VMEM((4, _CHW), jnp.float32),
            pltpu.VMEM((4, _CHW), jnp.float32),
            pltpu.SemaphoreType.DMA((4,)),
            pltpu.SemaphoreType.DMA((4,)),
            pltpu.SemaphoreType.DMA((4,)),
        ],
    )(_sc_body)
    out = run(vm2, j2)
    return out.reshape(1, 1, _NBL, _NT, _NF)


# speedup vs baseline: 1.3771x; 1.3771x over previous
import functools

import jax
import jax.numpy as jnp
from jax import lax
from jax.experimental import pallas as pl
from jax.experimental.pallas import tpu as pltpu
from jax.experimental.pallas import tpu_sc as plsc

_NBL = 64
_NANT = 65
_NT = 128
_NF = 4096
_COLS = _NT * _NF

_T_TC = 112
_T_SC = _NT - _T_TC
_CT = 8

_SC_COL0 = _T_TC * _NF
_SC_COLS = _T_SC * _NF
_NW = 32
_CW = _SC_COLS // _NW
_L = 16


def _tc_body(vm_ref, j_ref, out_ref):
    out_ref[...] = j_ref[0:_NBL] * vm_ref[...] * j_ref[1:_NANT]


def _mul3(dst, a, b, c, n):
    @plsc.parallel_loop(0, n, step=_L, unroll=8)
    def _body(i):
        sl = pl.ds(i, _L)
        dst[sl] = a[sl] * b[sl] * c[sl]


def _sc_body(vm_hbm, j_hbm, out_hbm, jb, vmb, ob, jsem, vsem, osem):
    c = lax.axis_index("c")
    s = lax.axis_index("s")
    wid = s * 2 + c
    cb = _SC_COL0 + wid * _CW

    def jsrc(r):
        return j_hbm.at[r, pl.ds(cb, _CW)]

    def vsrc(r):
        return vm_hbm.at[r, pl.ds(cb, _CW)]

    def odst(r):
        return out_hbm.at[r, pl.ds(wid * _CW, _CW)]

    pltpu.sync_copy(jsrc(0), jb.at[0])
    pltpu.async_copy(jsrc(1), jb.at[1], jsem.at[1])
    pltpu.async_copy(jsrc(2), jb.at[2], jsem.at[2])
    pltpu.async_copy(vsrc(0), vmb.at[0], vsem.at[0])
    pltpu.async_copy(vsrc(1), vmb.at[1], vsem.at[1])

    def block(k, _):
        for q in range(4):
            r = 4 * k + q
            jL = jb.at[q]
            jR = jb.at[(q + 1) % 4]
            vcur = vmb.at[q]
            ocur = ob.at[q]

            @pl.when(r <= _NANT - 4)
            def _pj():
                pltpu.async_copy(jsrc(r + 3), jb.at[(q + 3) % 4],
                                 jsem.at[(q + 3) % 4])

            @pl.when(r <= _NBL - 3)
            def _pv():
                pltpu.async_copy(vsrc(r + 2), vmb.at[(q + 2) % 4],
                                 vsem.at[(q + 2) % 4])

            pltpu.make_async_copy(jsrc(r + 1), jR,
                                  jsem.at[(q + 1) % 4]).wait()
            pltpu.make_async_copy(vsrc(r), vcur, vsem.at[q]).wait()

            @pl.when(r >= 4)
            def _po():
                pltpu.make_async_copy(ocur, odst(r), osem.at[q]).wait()

            _mul3(ocur, jL, vcur, jR, _CW)
            pltpu.async_copy(ocur, odst(r), osem.at[q])
        return 0

    lax.fori_loop(0, _NBL // 4, block, 0)
    for q in range(4):
        pltpu.make_async_copy(ob.at[q], odst(q), osem.at[q]).wait()


def kernel(V_m, jones):
    vm3 = V_m.reshape(_NBL, _NT, _NF)
    j3 = jones.reshape(_NANT, _NT, _NF)

    vm2 = V_m.reshape(_NBL, _COLS)
    j2 = jones.reshape(_NANT, _COLS)
    mesh = plsc.VectorSubcoreMesh(core_axis_name="c", subcore_axis_name="s")
    sc_run = functools.partial(
        pl.kernel,
        mesh=mesh,
        out_type=jax.ShapeDtypeStruct((_NBL, _SC_COLS), jnp.float32),
        scratch_types=[
            pltpu.VMEM((4, _CW), jnp.float32),
            pltpu.VMEM((4, _CW), jnp.float32),
            pltpu.VMEM((4, _CW), jnp.float32),
            pltpu.SemaphoreType.DMA((4,)),
            pltpu.SemaphoreType.DMA((4,)),
            pltpu.SemaphoreType.DMA((4,)),
        ],
    )(_sc_body)
    sc_out = sc_run(vm2, j2)

    tc_out = pl.pallas_call(
        _tc_body,
        grid=(_T_TC // _CT,),
        in_specs=[
            pl.BlockSpec((_NBL, _CT, _NF), lambda i: (0, i, 0)),
            pl.BlockSpec((_NANT, _CT, _NF), lambda i: (0, i, 0)),
        ],
        out_specs=pl.BlockSpec((_NBL, _CT, _NF), lambda i: (0, i, 0)),
        out_shape=jax.ShapeDtypeStruct((_NBL, _NT, _NF), jnp.float32),
    )(vm3, j3)

    out = lax.dynamic_update_slice(
        tc_out, sc_out.reshape(_NBL, _T_SC, _NF), (0, _T_TC, 0))
    return out.reshape(1, 1, _NBL, _NT, _NF)
